# Initial kernel scaffold; baseline (speedup 1.0000x reference)
import jax
import jax.numpy as jnp
from jax.experimental import pallas as pl


def kernel(features, edge_index, W0, b0, W1, b1):
    raise NotImplementedError("write your pallas kernel here")


# speedup vs baseline: 12.8387x; 12.8387x over previous
import functools

import jax
import jax.numpy as jnp
from jax import lax
from jax.experimental import pallas as pl
from jax.experimental.pallas import tpu as pltpu
from jax.experimental.pallas import tpu_sc as plsc

NC = 2
NS = 16
NW = NC * NS
CH = 128
F = 16


def _sc_mesh():
    return plsc.VectorSubcoreMesh(
        core_axis_name="c", subcore_axis_name="s",
        num_cores=NC, num_subcores=NS)


def _make_deg_kernel(NPAD, CPW):
    stripe = NPAD // NS

    def body(dsts_hbm, zeros_hbm, ones_hbm, out_hbm, dst_v, rows_v, agg_sh, sem):
        c = lax.axis_index("c")
        s = lax.axis_index("s")
        wid = s * NC + c
        pltpu.sync_copy(zeros_hbm.at[pl.ds(s * stripe, stripe)],
                        agg_sh.at[pl.ds(s * stripe, stripe)])
        pltpu.sync_copy(ones_hbm, rows_v)
        pltpu.sync_copy(dsts_hbm.at[wid], dst_v)
        plsc.subcore_barrier()

        def step(j, carry):
            pltpu.sync_copy(rows_v, agg_sh.at[dst_v.at[j]], add=True)
            return carry

        lax.fori_loop(0, CPW, step, 0)
        plsc.subcore_barrier()
        pltpu.sync_copy(agg_sh.at[pl.ds(s * stripe, stripe)],
                        out_hbm.at[pl.ds(c * NPAD + s * stripe, stripe)])

    return pl.kernel(
        body,
        out_type=jax.ShapeDtypeStruct((2 * NPAD, F), jnp.float32),
        mesh=_sc_mesh(),
        compiler_params=pltpu.CompilerParams(use_tc_tiling_on_sc=False),
        scratch_types=[
            pltpu.VMEM((CPW, CH), jnp.int32),
            pltpu.VMEM((CH, F), jnp.float32),
            pltpu.VMEM_SHARED((NPAD, F), jnp.float32),
            pltpu.SemaphoreType.DMA,
        ],
    )


def _make_agg_kernel(NPAD, CPW):
    stripe = NPAD // NS

    def body(h_hbm, srcs_hbm, dsts_hbm, zeros_hbm, out_hbm,
             src_v, dst_v, rows_v, agg_sh, sem):
        c = lax.axis_index("c")
        s = lax.axis_index("s")
        wid = s * NC + c
        pltpu.sync_copy(zeros_hbm.at[pl.ds(s * stripe, stripe)],
                        agg_sh.at[pl.ds(s * stripe, stripe)])
        pltpu.sync_copy(srcs_hbm.at[wid], src_v)
        pltpu.sync_copy(dsts_hbm.at[wid], dst_v)
        plsc.subcore_barrier()

        def step(j, carry):
            pltpu.async_copy(h_hbm.at[src_v.at[j]], rows_v, sem).wait()
            pltpu.sync_copy(rows_v, agg_sh.at[dst_v.at[j]], add=True)
            return carry

        lax.fori_loop(0, CPW, step, 0)
        plsc.subcore_barrier()
        pltpu.sync_copy(agg_sh.at[pl.ds(s * stripe, stripe)],
                        out_hbm.at[pl.ds(c * NPAD + s * stripe, stripe)])

    return pl.kernel(
        body,
        out_type=jax.ShapeDtypeStruct((2 * NPAD, F), jnp.float32),
        mesh=_sc_mesh(),
        compiler_params=pltpu.CompilerParams(use_tc_tiling_on_sc=False),
        scratch_types=[
            pltpu.VMEM((CPW, CH), jnp.int32),
            pltpu.VMEM((CPW, CH), jnp.int32),
            pltpu.VMEM((CH, F), jnp.float32),
            pltpu.VMEM_SHARED((NPAD, F), jnp.float32),
            pltpu.SemaphoreType.DMA,
        ],
    )


def kernel(features, edge_index, W0, b0, W1, b1):
    N, IN_FEATS = features.shape
    E = edge_index.shape[1]
    f32 = jnp.float32

    NPAD = ((N // 256) + 1) * 256
    CPW = -(-E // (NW * CH))
    EP = NW * CH * CPW

    src = edge_index[0]
    dst = edge_index[1]
    srcs = jnp.concatenate(
        [src, jnp.zeros((EP - E,), jnp.int32)]).reshape(NW, CPW, CH)
    dsts = jnp.concatenate(
        [dst, jnp.full((EP - E,), N, jnp.int32)]).reshape(NW, CPW, CH)

    zeros_pad = jnp.zeros((NPAD, F), f32)
    ones_rows = jnp.ones((CH, F), f32)

    deg_k = _make_deg_kernel(NPAD, CPW)
    agg_k = _make_agg_kernel(NPAD, CPW)

    degp = deg_k(dsts, zeros_pad, ones_rows)

    def tc1(feat_ref, w0_ref, degp_ref, h0n_ref, norm_ref):
        degp2 = degp_ref[...]
        deg = degp2[:NPAD] + degp2[NPAD:]
        norm = lax.rsqrt(jnp.maximum(deg, 1.0))
        norm_ref[...] = norm
        h0n_ref[...] = jnp.dot(feat_ref[...], w0_ref[...],
                               preferred_element_type=f32) * norm[:N]

    h0n, norm = pl.pallas_call(
        tc1,
        out_shape=[jax.ShapeDtypeStruct((N, F), f32),
                   jax.ShapeDtypeStruct((NPAD, F), f32)],
    )(features, W0, degp)

    agg0p = agg_k(h0n, srcs, dsts, zeros_pad)

    def tc2(aggp_ref, norm_ref, w1_ref, b0_ref, h1n_ref):
        aggp2 = aggp_ref[...]
        agg0 = (aggp2[:NPAD] + aggp2[NPAD:])[:N]
        nrm = norm_ref[...][:N]
        z0 = jnp.maximum(agg0 * nrm + b0_ref[...], 0.0)
        h1n_ref[...] = jnp.dot(z0, w1_ref[...],
                               preferred_element_type=f32) * nrm

    h1n = pl.pallas_call(
        tc2,
        out_shape=jax.ShapeDtypeStruct((N, F), f32),
    )(agg0p, norm, W1, b0.reshape(1, F))

    agg1p = agg_k(h1n, srcs, dsts, zeros_pad)

    def tc3(aggp_ref, norm_ref, b1_ref, out_ref):
        aggp2 = aggp_ref[...]
        agg1 = (aggp2[:NPAD] + aggp2[NPAD:])[:N]
        out_ref[...] = agg1 * norm_ref[...][:N] + b1_ref[...]

    out = pl.pallas_call(
        tc3,
        out_shape=jax.ShapeDtypeStruct((N, F), f32),
    )(agg1p, norm, b1.reshape(1, F))

    return out


# speedup vs baseline: 15.0956x; 1.1758x over previous
import functools

import jax
import jax.numpy as jnp
from jax import lax
from jax.experimental import pallas as pl
from jax.experimental.pallas import tpu as pltpu
from jax.experimental.pallas import tpu_sc as plsc

NC = 2
NS = 16
NW = NC * NS
CH = 128
F = 16
NB = 8
LA = 4


def _sc_mesh():
    return plsc.VectorSubcoreMesh(
        core_axis_name="c", subcore_axis_name="s",
        num_cores=NC, num_subcores=NS)


def _make_deg_kernel(NPAD, CPW):
    stripe = NPAD // NS
    n_groups = CPW // NB

    def body(dsts_hbm, zeros_hbm, ones_hbm, out_hbm, dst_v, rows_v, agg_sh, ssem):
        c = lax.axis_index("c")
        s = lax.axis_index("s")
        wid = s * NC + c
        pltpu.sync_copy(zeros_hbm.at[pl.ds(s * stripe, stripe)],
                        agg_sh.at[pl.ds(s * stripe, stripe)])
        pltpu.sync_copy(ones_hbm, rows_v)
        pltpu.sync_copy(dsts_hbm.at[wid], dst_v)
        plsc.subcore_barrier()

        def fire(j, t):
            pltpu.async_copy(rows_v, agg_sh.at[dst_v.at[j]], ssem.at[t],
                             add=True)

        def drain(t):
            pltpu.make_async_copy(rows_v, agg_sh.at[dst_v.at[0]],
                                  ssem.at[t]).wait()

        def group(g, carry):
            for t in range(NB):
                j = g * NB + t
                @pl.when(j >= NB)
                def _():
                    drain(t)
                fire(j, t)
            return carry

        lax.fori_loop(0, n_groups, group, 0)
        for t in range(NB):
            drain(t)
        plsc.subcore_barrier()
        pltpu.sync_copy(agg_sh.at[pl.ds(s * stripe, stripe)],
                        out_hbm.at[pl.ds(c * NPAD + s * stripe, stripe)])

    return pl.kernel(
        body,
        out_type=jax.ShapeDtypeStruct((2 * NPAD, F), jnp.float32),
        mesh=_sc_mesh(),
        compiler_params=pltpu.CompilerParams(use_tc_tiling_on_sc=False),
        scratch_types=[
            pltpu.VMEM((CPW, CH), jnp.int32),
            pltpu.VMEM((CH, F), jnp.float32),
            pltpu.VMEM_SHARED((NPAD, F), jnp.float32),
            pltpu.SemaphoreType.DMA((NB,)),
        ],
    )


def _make_agg_kernel(NPAD, CPW):
    stripe = NPAD // NS
    n_groups = CPW // NB

    def body(h_hbm, srcs_hbm, dsts_hbm, zeros_hbm, out_hbm,
             src_v, dst_v, rows_v, agg_sh, gsem, ssem):
        c = lax.axis_index("c")
        s = lax.axis_index("s")
        wid = s * NC + c
        pltpu.sync_copy(zeros_hbm.at[pl.ds(s * stripe, stripe)],
                        agg_sh.at[pl.ds(s * stripe, stripe)])
        pltpu.sync_copy(srcs_hbm.at[wid], src_v)
        pltpu.sync_copy(dsts_hbm.at[wid], dst_v)
        plsc.subcore_barrier()

        def fire_gather(j, t):
            pltpu.async_copy(h_hbm.at[src_v.at[j]], rows_v.at[t], gsem.at[t])

        def wait_gather(t):
            pltpu.make_async_copy(h_hbm.at[src_v.at[0]], rows_v.at[t],
                                  gsem.at[t]).wait()

        def fire_scatter(j, t):
            pltpu.async_copy(rows_v.at[t], agg_sh.at[dst_v.at[j]], ssem.at[t],
                             add=True)

        def wait_scatter(t):
            pltpu.make_async_copy(rows_v.at[t], agg_sh.at[dst_v.at[0]],
                                  ssem.at[t]).wait()

        for t in range(LA):
            fire_gather(t, t)

        def group(g, carry):
            for t in range(NB):
                j = g * NB + t
                wait_gather(t)
                fire_scatter(j, t)
                s2 = (t + LA) % NB
                jg = j + LA

                @pl.when(jnp.logical_and(jg >= NB, jg < CPW))
                def _():
                    wait_scatter(s2)

                @pl.when(jg < CPW)
                def _():
                    fire_gather(jg, s2)
            return carry

        lax.fori_loop(0, n_groups, group, 0)
        for t in range(NB):
            wait_scatter(t)
        plsc.subcore_barrier()
        pltpu.sync_copy(agg_sh.at[pl.ds(s * stripe, stripe)],
                        out_hbm.at[pl.ds(c * NPAD + s * stripe, stripe)])

    return pl.kernel(
        body,
        out_type=jax.ShapeDtypeStruct((2 * NPAD, F), jnp.float32),
        mesh=_sc_mesh(),
        compiler_params=pltpu.CompilerParams(use_tc_tiling_on_sc=False),
        scratch_types=[
            pltpu.VMEM((CPW, CH), jnp.int32),
            pltpu.VMEM((CPW, CH), jnp.int32),
            pltpu.VMEM((NB, CH, F), jnp.float32),
            pltpu.VMEM_SHARED((NPAD, F), jnp.float32),
            pltpu.SemaphoreType.DMA((NB,)),
            pltpu.SemaphoreType.DMA((NB,)),
        ],
    )


def kernel(features, edge_index, W0, b0, W1, b1):
    N, IN_FEATS = features.shape
    E = edge_index.shape[1]
    f32 = jnp.float32

    NPAD = ((N // 256) + 1) * 256
    CPW = -(-E // (NW * CH))
    CPW = -(-CPW // NB) * NB
    EP = NW * CH * CPW

    src = edge_index[0]
    dst = edge_index[1]
    srcs = jnp.concatenate(
        [src, jnp.zeros((EP - E,), jnp.int32)]).reshape(NW, CPW, CH)
    dsts = jnp.concatenate(
        [dst, jnp.full((EP - E,), N, jnp.int32)]).reshape(NW, CPW, CH)

    zeros_pad = jnp.zeros((NPAD, F), f32)
    ones_rows = jnp.ones((CH, F), f32)

    deg_k = _make_deg_kernel(NPAD, CPW)
    agg_k = _make_agg_kernel(NPAD, CPW)

    degp = deg_k(dsts, zeros_pad, ones_rows)

    def tc1(feat_ref, w0_ref, degp_ref, h0n_ref, norm_ref):
        degp2 = degp_ref[...]
        deg = degp2[:NPAD] + degp2[NPAD:]
        norm = lax.rsqrt(jnp.maximum(deg, 1.0))
        norm_ref[...] = norm
        h0n_ref[...] = jnp.dot(feat_ref[...], w0_ref[...],
                               preferred_element_type=f32) * norm[:N]

    h0n, norm = pl.pallas_call(
        tc1,
        out_shape=[jax.ShapeDtypeStruct((N, F), f32),
                   jax.ShapeDtypeStruct((NPAD, F), f32)],
    )(features, W0, degp)

    agg0p = agg_k(h0n, srcs, dsts, zeros_pad)

    def tc2(aggp_ref, norm_ref, w1_ref, b0_ref, h1n_ref):
        aggp2 = aggp_ref[...]
        agg0 = (aggp2[:NPAD] + aggp2[NPAD:])[:N]
        nrm = norm_ref[...][:N]
        z0 = jnp.maximum(agg0 * nrm + b0_ref[...], 0.0)
        h1n_ref[...] = jnp.dot(z0, w1_ref[...],
                               preferred_element_type=f32) * nrm

    h1n = pl.pallas_call(
        tc2,
        out_shape=jax.ShapeDtypeStruct((N, F), f32),
    )(agg0p, norm, W1, b0.reshape(1, F))

    agg1p = agg_k(h1n, srcs, dsts, zeros_pad)

    def tc3(aggp_ref, norm_ref, b1_ref, out_ref):
        aggp2 = aggp_ref[...]
        agg1 = (aggp2[:NPAD] + aggp2[NPAD:])[:N]
        out_ref[...] = agg1 * norm_ref[...][:N] + b1_ref[...]

    out = pl.pallas_call(
        tc3,
        out_shape=jax.ShapeDtypeStruct((N, F), f32),
    )(agg1p, norm, b1.reshape(1, F))

    return out


# speedup vs baseline: 15.3491x; 1.0168x over previous
import functools

import jax
import jax.numpy as jnp
from jax import lax
from jax.experimental import pallas as pl
from jax.experimental.pallas import tpu as pltpu
from jax.experimental.pallas import tpu_sc as plsc

NC = 2
NS = 16
NW = NC * NS
CH = 128
F = 16
NB = 8
LA = 4


def _sc_mesh():
    return plsc.VectorSubcoreMesh(
        core_axis_name="c", subcore_axis_name="s",
        num_cores=NC, num_subcores=NS)


def _make_deg_kernel(NPAD, CPW):
    stripe = NPAD // NS
    n_groups = CPW // NB

    def body(dsts_hbm, zeros_hbm, ones_hbm, out_hbm, dst_v, rows_v, agg_sh, ssem):
        c = lax.axis_index("c")
        s = lax.axis_index("s")
        wid = s * NC + c
        pltpu.sync_copy(zeros_hbm.at[pl.ds(s * stripe, stripe)],
                        agg_sh.at[pl.ds(s * stripe, stripe)])
        pltpu.sync_copy(ones_hbm, rows_v)
        pltpu.sync_copy(dsts_hbm.at[wid], dst_v)
        plsc.subcore_barrier()

        def fire(j, t):
            pltpu.async_copy(rows_v, agg_sh.at[dst_v.at[j]], ssem.at[t],
                             add=True)

        def drain(t):
            pltpu.make_async_copy(rows_v, agg_sh.at[dst_v.at[0]],
                                  ssem.at[t]).wait()

        def group(g, carry):
            for t in range(NB):
                j = g * NB + t
                @pl.when(j >= NB)
                def _():
                    drain(t)
                fire(j, t)
            return carry

        lax.fori_loop(0, n_groups, group, 0)
        for t in range(NB):
            drain(t)
        plsc.subcore_barrier()
        pltpu.sync_copy(agg_sh.at[pl.ds(s * stripe, stripe)],
                        out_hbm.at[pl.ds(c * NPAD + s * stripe, stripe)])

    return pl.kernel(
        body,
        out_type=jax.ShapeDtypeStruct((2 * NPAD, F), jnp.float32),
        mesh=_sc_mesh(),
        compiler_params=pltpu.CompilerParams(use_tc_tiling_on_sc=False),
        scratch_types=[
            pltpu.VMEM((CPW, CH), jnp.int32),
            pltpu.VMEM((CH, F), jnp.float32),
            pltpu.VMEM_SHARED((NPAD, F), jnp.float32),
            pltpu.SemaphoreType.DMA((NB,)),
        ],
    )


def _make_agg_kernel(NPAD, CPW):
    stripe = NPAD // NS
    n_groups = CPW // NB

    def body(h_hbm, srcs_hbm, dsts_hbm, zeros_hbm, out_hbm,
             src_v, dst_v, rows_v, agg_sh, gsem, ssem):
        c = lax.axis_index("c")
        s = lax.axis_index("s")
        wid = s * NC + c
        pltpu.sync_copy(zeros_hbm.at[pl.ds(s * stripe, stripe)],
                        agg_sh.at[pl.ds(s * stripe, stripe)])
        pltpu.sync_copy(srcs_hbm.at[wid], src_v)
        pltpu.sync_copy(dsts_hbm.at[wid], dst_v)
        plsc.subcore_barrier()

        def fire_gather(j, t):
            pltpu.async_copy(h_hbm.at[src_v.at[j]], rows_v.at[t], gsem.at[t])

        def wait_gather(t):
            pltpu.make_async_copy(h_hbm.at[src_v.at[0]], rows_v.at[t],
                                  gsem.at[t]).wait()

        def fire_scatter(j, t):
            pltpu.async_copy(rows_v.at[t], agg_sh.at[dst_v.at[j]], ssem.at[t],
                             add=True)

        def wait_scatter(t):
            pltpu.make_async_copy(rows_v.at[t], agg_sh.at[dst_v.at[0]],
                                  ssem.at[t]).wait()

        for t in range(LA):
            fire_gather(t, t)

        def group(g, carry):
            for t in range(NB):
                j = g * NB + t
                wait_gather(t)
                fire_scatter(j, t)
                s2 = (t + LA) % NB
                jg = j + LA

                @pl.when(jnp.logical_and(jg >= NB, jg < CPW))
                def _():
                    wait_scatter(s2)

                @pl.when(jg < CPW)
                def _():
                    fire_gather(jg, s2)
            return carry

        lax.fori_loop(0, n_groups, group, 0)
        for t in range(NB):
            wait_scatter(t)
        plsc.subcore_barrier()
        pltpu.sync_copy(agg_sh.at[pl.ds(s * stripe, stripe)],
                        out_hbm.at[pl.ds(c * NPAD + s * stripe, stripe)])

    return pl.kernel(
        body,
        out_type=jax.ShapeDtypeStruct((2 * NPAD, F), jnp.float32),
        mesh=_sc_mesh(),
        compiler_params=pltpu.CompilerParams(use_tc_tiling_on_sc=False),
        scratch_types=[
            pltpu.VMEM((CPW, CH), jnp.int32),
            pltpu.VMEM((CPW, CH), jnp.int32),
            pltpu.VMEM((NB, CH, F), jnp.float32),
            pltpu.VMEM_SHARED((NPAD, F), jnp.float32),
            pltpu.SemaphoreType.DMA((NB,)),
            pltpu.SemaphoreType.DMA((NB,)),
        ],
    )


def kernel(features, edge_index, W0, b0, W1, b1):
    N, IN_FEATS = features.shape
    E = edge_index.shape[1]
    f32 = jnp.float32

    NPAD = ((N // 256) + 1) * 256
    CPW = -(-E // (NW * CH))
    CPW = -(-CPW // NB) * NB
    EP = NW * CH * CPW

    src = edge_index[0]
    dst = edge_index[1]
    srcs = jnp.concatenate(
        [src, jnp.zeros((EP - E,), jnp.int32)]).reshape(NW, CPW, CH)
    pad_dst = N + jnp.arange(EP - E, dtype=jnp.int32) % (NPAD - N)
    dsts = jnp.concatenate([dst, pad_dst]).reshape(NW, CPW, CH)

    zeros_pad = jnp.zeros((NPAD, F), f32)
    ones_rows = jnp.ones((CH, F), f32)

    deg_k = _make_deg_kernel(NPAD, CPW)
    agg_k = _make_agg_kernel(NPAD, CPW)

    degp = deg_k(dsts, zeros_pad, ones_rows)

    def tc1(feat_ref, w0_ref, degp_ref, h0n_ref, norm_ref):
        degp2 = degp_ref[...]
        deg = degp2[:NPAD] + degp2[NPAD:]
        norm = lax.rsqrt(jnp.maximum(deg, 1.0))
        norm_ref[...] = norm
        h0n_ref[...] = jnp.dot(feat_ref[...], w0_ref[...],
                               preferred_element_type=f32) * norm[:N]

    h0n, norm = pl.pallas_call(
        tc1,
        out_shape=[jax.ShapeDtypeStruct((N, F), f32),
                   jax.ShapeDtypeStruct((NPAD, F), f32)],
    )(features, W0, degp)

    agg0p = agg_k(h0n, srcs, dsts, zeros_pad)

    def tc2(aggp_ref, norm_ref, w1_ref, b0_ref, h1n_ref):
        aggp2 = aggp_ref[...]
        agg0 = (aggp2[:NPAD] + aggp2[NPAD:])[:N]
        nrm = norm_ref[...][:N]
        z0 = jnp.maximum(agg0 * nrm + b0_ref[...], 0.0)
        h1n_ref[...] = jnp.dot(z0, w1_ref[...],
                               preferred_element_type=f32) * nrm

    h1n = pl.pallas_call(
        tc2,
        out_shape=jax.ShapeDtypeStruct((N, F), f32),
    )(agg0p, norm, W1, b0.reshape(1, F))

    agg1p = agg_k(h1n, srcs, dsts, zeros_pad)

    def tc3(aggp_ref, norm_ref, b1_ref, out_ref):
        aggp2 = aggp_ref[...]
        agg1 = (aggp2[:NPAD] + aggp2[NPAD:])[:N]
        out_ref[...] = agg1 * norm_ref[...][:N] + b1_ref[...]

    out = pl.pallas_call(
        tc3,
        out_shape=jax.ShapeDtypeStruct((N, F), f32),
    )(agg1p, norm, b1.reshape(1, F))

    return out


# speedup vs baseline: 22.3072x; 1.4533x over previous
import functools

import jax
import jax.numpy as jnp
from jax import lax
from jax.experimental import pallas as pl
from jax.experimental.pallas import tpu as pltpu
from jax.experimental.pallas import tpu_sc as plsc

NC = 2
NS = 16
NW = NC * NS
CH = 128
F = 16
NB = 8
LA = 4


def _sc_mesh():
    return plsc.VectorSubcoreMesh(
        core_axis_name="c", subcore_axis_name="s",
        num_cores=NC, num_subcores=NS)


def _make_deg_kernel(NPAD, CPW):
    stripe = NPAD // NS
    n_groups = CPW // NB

    def body(dsts_hbm, zeros_hbm, ones_hbm, out_hbm, dst_v, rows_v, agg_sh, ssem):
        c = lax.axis_index("c")
        s = lax.axis_index("s")
        wid = s * NC + c
        pltpu.sync_copy(zeros_hbm.at[pl.ds(s * stripe, stripe)],
                        agg_sh.at[pl.ds(s * stripe, stripe)])
        pltpu.sync_copy(ones_hbm, rows_v)
        pltpu.sync_copy(dsts_hbm.at[:, wid], dst_v)
        plsc.subcore_barrier()

        def fire(j, t):
            pltpu.async_copy(rows_v, agg_sh.at[dst_v.at[j]], ssem.at[t],
                             add=True)

        def drain(t):
            pltpu.make_async_copy(rows_v, agg_sh.at[dst_v.at[0]],
                                  ssem.at[t]).wait()

        def group(g, carry):
            for t in range(NB):
                j = g * NB + t
                @pl.when(j >= NB)
                def _():
                    drain(t)
                fire(j, t)
            return carry

        lax.fori_loop(0, n_groups, group, 0)
        for t in range(NB):
            drain(t)
        plsc.subcore_barrier()
        pltpu.sync_copy(agg_sh.at[pl.ds(s * stripe, stripe)],
                        out_hbm.at[pl.ds(c * NPAD + s * stripe, stripe)])

    return pl.kernel(
        body,
        out_type=jax.ShapeDtypeStruct((2 * NPAD, F), jnp.float32),
        mesh=_sc_mesh(),
        compiler_params=pltpu.CompilerParams(use_tc_tiling_on_sc=False),
        scratch_types=[
            pltpu.VMEM((CPW, CH), jnp.int32),
            pltpu.VMEM((CH, F), jnp.float32),
            pltpu.VMEM_SHARED((NPAD, F), jnp.float32),
            pltpu.SemaphoreType.DMA((NB,)),
        ],
    )


def _make_agg_kernel(NPAD, CPW):
    stripe = NPAD // NS
    n_groups = CPW // NB

    def body(h_hbm, srcs_hbm, dsts_hbm, zeros_hbm, out_hbm,
             src_v, dst_v, rows_v, agg_sh, gsem, ssem):
        c = lax.axis_index("c")
        s = lax.axis_index("s")
        wid = s * NC + c
        pltpu.sync_copy(zeros_hbm.at[pl.ds(s * stripe, stripe)],
                        agg_sh.at[pl.ds(s * stripe, stripe)])
        pltpu.sync_copy(srcs_hbm.at[:, wid], src_v)
        pltpu.sync_copy(dsts_hbm.at[:, wid], dst_v)
        plsc.subcore_barrier()

        def fire_gather(j, t):
            pltpu.async_copy(h_hbm.at[src_v.at[j]], rows_v.at[t], gsem.at[t])

        def wait_gather(t):
            pltpu.make_async_copy(h_hbm.at[src_v.at[0]], rows_v.at[t],
                                  gsem.at[t]).wait()

        def fire_scatter(j, t):
            pltpu.async_copy(rows_v.at[t], agg_sh.at[dst_v.at[j]], ssem.at[t],
                             add=True)

        def wait_scatter(t):
            pltpu.make_async_copy(rows_v.at[t], agg_sh.at[dst_v.at[0]],
                                  ssem.at[t]).wait()

        for t in range(LA):
            fire_gather(t, t)

        def group(g, carry):
            for t in range(NB):
                j = g * NB + t
                wait_gather(t)
                fire_scatter(j, t)
                s2 = (t + LA) % NB
                jg = j + LA

                @pl.when(jnp.logical_and(jg >= NB, jg < CPW))
                def _():
                    wait_scatter(s2)

                @pl.when(jg < CPW)
                def _():
                    fire_gather(jg, s2)
            return carry

        lax.fori_loop(0, n_groups, group, 0)
        for t in range(NB):
            wait_scatter(t)
        plsc.subcore_barrier()
        pltpu.sync_copy(agg_sh.at[pl.ds(s * stripe, stripe)],
                        out_hbm.at[pl.ds(c * NPAD + s * stripe, stripe)])

    return pl.kernel(
        body,
        out_type=jax.ShapeDtypeStruct((2 * NPAD, F), jnp.float32),
        mesh=_sc_mesh(),
        compiler_params=pltpu.CompilerParams(use_tc_tiling_on_sc=False),
        scratch_types=[
            pltpu.VMEM((CPW, CH), jnp.int32),
            pltpu.VMEM((CPW, CH), jnp.int32),
            pltpu.VMEM((NB, CH, F), jnp.float32),
            pltpu.VMEM_SHARED((NPAD, F), jnp.float32),
            pltpu.SemaphoreType.DMA((NB,)),
            pltpu.SemaphoreType.DMA((NB,)),
        ],
    )


def kernel(features, edge_index, W0, b0, W1, b1):
    N, IN_FEATS = features.shape
    E = edge_index.shape[1]
    f32 = jnp.float32

    NPAD = ((N // 256) + 1) * 256
    CPW = -(-E // (NW * CH))
    CPW = -(-CPW // NB) * NB
    EP = NW * CH * CPW

    src = edge_index[0]
    dst = edge_index[1]
    pad_i = jnp.arange(EP - E, dtype=jnp.int32)
    srcs = jnp.concatenate([src, pad_i % 128]).reshape(CPW, NW, CH)
    dsts = jnp.concatenate([dst, N + pad_i % (NPAD - N)]).reshape(CPW, NW, CH)

    zeros_pad = jnp.zeros((NPAD, F), f32)
    ones_rows = jnp.ones((CH, F), f32)

    deg_k = _make_deg_kernel(NPAD, CPW)
    agg_k = _make_agg_kernel(NPAD, CPW)

    degp = deg_k(dsts, zeros_pad, ones_rows)

    def tc1(feat_ref, w0_ref, degp_ref, h0n_ref, norm_ref):
        degp2 = degp_ref[...]
        deg = degp2[:NPAD] + degp2[NPAD:]
        norm = lax.rsqrt(jnp.maximum(deg, 1.0))
        norm_ref[...] = norm
        h0n_ref[...] = jnp.dot(feat_ref[...], w0_ref[...],
                               preferred_element_type=f32) * norm[:N]

    h0n, norm = pl.pallas_call(
        tc1,
        out_shape=[jax.ShapeDtypeStruct((N, F), f32),
                   jax.ShapeDtypeStruct((NPAD, F), f32)],
    )(features, W0, degp)

    agg0p = agg_k(h0n, srcs, dsts, zeros_pad)

    def tc2(aggp_ref, norm_ref, w1_ref, b0_ref, h1n_ref):
        aggp2 = aggp_ref[...]
        agg0 = (aggp2[:NPAD] + aggp2[NPAD:])[:N]
        nrm = norm_ref[...][:N]
        z0 = jnp.maximum(agg0 * nrm + b0_ref[...], 0.0)
        h1n_ref[...] = jnp.dot(z0, w1_ref[...],
                               preferred_element_type=f32) * nrm

    h1n = pl.pallas_call(
        tc2,
        out_shape=jax.ShapeDtypeStruct((N, F), f32),
    )(agg0p, norm, W1, b0.reshape(1, F))

    agg1p = agg_k(h1n, srcs, dsts, zeros_pad)

    def tc3(aggp_ref, norm_ref, b1_ref, out_ref):
        aggp2 = aggp_ref[...]
        agg1 = (aggp2[:NPAD] + aggp2[NPAD:])[:N]
        out_ref[...] = agg1 * norm_ref[...][:N] + b1_ref[...]

    out = pl.pallas_call(
        tc3,
        out_shape=jax.ShapeDtypeStruct((N, F), f32),
    )(agg1p, norm, b1.reshape(1, F))

    return out


# speedup vs baseline: 29.3420x; 1.3154x over previous
import functools

import jax
import jax.numpy as jnp
from jax import lax
from jax.experimental import pallas as pl
from jax.experimental.pallas import tpu as pltpu
from jax.experimental.pallas import tpu_sc as plsc

NC = 2
NS = 16
NW = NC * NS
CH = 128
F = 16
NB = 8
LA = 4


def _sc_mesh():
    return plsc.VectorSubcoreMesh(
        core_axis_name="c", subcore_axis_name="s",
        num_cores=NC, num_subcores=NS)


def _make_deg_kernel(NPAD, CPW):
    stripe = NPAD // NS
    n_groups = CPW // NB

    def body(dsts_hbm, zeros_hbm, ones_hbm, out_hbm, dst_v, rows_v, agg_sh, ssem):
        c = lax.axis_index("c")
        s = lax.axis_index("s")
        wid = s * NC + c
        pltpu.sync_copy(zeros_hbm.at[pl.ds(s * stripe, stripe)],
                        agg_sh.at[pl.ds(s * stripe, stripe)])
        pltpu.sync_copy(ones_hbm, rows_v)
        pltpu.sync_copy(dsts_hbm.at[:, wid], dst_v)
        plsc.subcore_barrier()

        def fire(j, t):
            pltpu.async_copy(rows_v, agg_sh.at[dst_v.at[j]], ssem.at[t],
                             add=True)

        def drain(t):
            pltpu.make_async_copy(rows_v, agg_sh.at[dst_v.at[0]],
                                  ssem.at[t]).wait()

        def group(g, carry):
            for t in range(NB):
                j = g * NB + t
                @pl.when(j >= NB)
                def _():
                    drain(t)
                fire(j, t)
            return carry

        lax.fori_loop(0, n_groups, group, 0)
        for t in range(NB):
            drain(t)
        plsc.subcore_barrier()
        pltpu.sync_copy(agg_sh.at[pl.ds(s * stripe, stripe)],
                        out_hbm.at[pl.ds(c * NPAD + s * stripe, stripe)])

    return pl.kernel(
        body,
        out_type=jax.ShapeDtypeStruct((2 * NPAD, F), jnp.float32),
        mesh=_sc_mesh(),
        compiler_params=pltpu.CompilerParams(use_tc_tiling_on_sc=False),
        scratch_types=[
            pltpu.VMEM((CPW, CH), jnp.int32),
            pltpu.VMEM((CH, F), jnp.float32),
            pltpu.VMEM_SHARED((NPAD, F), jnp.float32),
            pltpu.SemaphoreType.DMA((NB,)),
        ],
    )


def _make_agg_kernel(NPAD, CPW):
    stripe = NPAD // NS
    n_groups = CPW // NB

    def body(h_hbm, srcs_hbm, dsts_hbm, zeros_hbm, out_hbm,
             src_v, dst_v, rows_v, agg_sh, gsem, ssem):
        c = lax.axis_index("c")
        s = lax.axis_index("s")
        wid = s * NC + c
        pltpu.sync_copy(zeros_hbm.at[pl.ds(s * stripe, stripe)],
                        agg_sh.at[pl.ds(s * stripe, stripe)])
        pltpu.sync_copy(srcs_hbm.at[:, wid], src_v)
        pltpu.sync_copy(dsts_hbm.at[:, wid], dst_v)
        plsc.subcore_barrier()

        def fire_gather(j, t):
            pltpu.async_copy(h_hbm.at[src_v.at[j]], rows_v.at[t], gsem.at[t])

        def wait_gather(t):
            pltpu.make_async_copy(h_hbm.at[src_v.at[0]], rows_v.at[t],
                                  gsem.at[t]).wait()

        def fire_scatter(j, t):
            pltpu.async_copy(rows_v.at[t], agg_sh.at[dst_v.at[j]], ssem.at[t],
                             add=True)

        def wait_scatter(t):
            pltpu.make_async_copy(rows_v.at[t], agg_sh.at[dst_v.at[0]],
                                  ssem.at[t]).wait()

        for t in range(LA):
            fire_gather(t, t)

        def group(g, carry):
            for t in range(NB):
                j = g * NB + t
                wait_gather(t)
                fire_scatter(j, t)
                s2 = (t + LA) % NB
                jg = j + LA

                @pl.when(jnp.logical_and(jg >= NB, jg < CPW))
                def _():
                    wait_scatter(s2)

                @pl.when(jg < CPW)
                def _():
                    fire_gather(jg, s2)
            return carry

        lax.fori_loop(0, n_groups, group, 0)
        for t in range(NB):
            wait_scatter(t)
        plsc.subcore_barrier()
        pltpu.sync_copy(agg_sh.at[pl.ds(s * stripe, stripe)],
                        out_hbm.at[pl.ds(c * NPAD + s * stripe, stripe)])

    return pl.kernel(
        body,
        out_type=jax.ShapeDtypeStruct((2 * NPAD, F), jnp.float32),
        mesh=_sc_mesh(),
        compiler_params=pltpu.CompilerParams(use_tc_tiling_on_sc=False),
        scratch_types=[
            pltpu.VMEM((CPW, CH), jnp.int32),
            pltpu.VMEM((CPW, CH), jnp.int32),
            pltpu.VMEM((NB, CH, F), jnp.float32),
            pltpu.VMEM_SHARED((NPAD, F), jnp.float32),
            pltpu.SemaphoreType.DMA((NB,)),
            pltpu.SemaphoreType.DMA((NB,)),
        ],
    )


def kernel(features, edge_index, W0, b0, W1, b1):
    N, IN_FEATS = features.shape
    E = edge_index.shape[1]
    f32 = jnp.float32

    NPAD = ((N // 256) + 1) * 256
    CPW = -(-E // (NW * CH))
    CPW = -(-CPW // NB) * NB
    EP = NW * CH * CPW
    P = N // 8
    PP = NPAD // 8

    src = edge_index[0]
    dst = edge_index[1]
    pad_i = jnp.arange(EP - E, dtype=jnp.int32)
    srcs = jnp.concatenate([src, pad_i % 128]).reshape(CPW, NW, CH)
    dsts = jnp.concatenate([dst, N + pad_i % (NPAD - N)]).reshape(CPW, NW, CH)

    zeros_pad = jnp.zeros((NPAD, F), f32)
    ones_rows = jnp.ones((CH, F), f32)

    deg_k = _make_deg_kernel(NPAD, CPW)
    agg_k = _make_agg_kernel(NPAD, CPW)

    degp = deg_k(dsts, zeros_pad, ones_rows)

    feats3 = features.reshape(P, 8, IN_FEATS)

    def tc_x0(f3_ref, w0_ref, x0_ref):
        parts = [jnp.dot(f3_ref[:, u, :], w0_ref[...],
                         preferred_element_type=f32) for u in range(8)]
        x0 = jnp.concatenate(parts, axis=1)
        x0_ref[...] = jnp.concatenate(
            [x0, jnp.zeros((PP - P, 8 * F), f32)], axis=0)

    x0_pack = pl.pallas_call(
        tc_x0,
        out_shape=jax.ShapeDtypeStruct((PP, 8 * F), f32),
    )(feats3, W0)

    def tc1(x0_ref, degp_ref, h0n_ref, norm_ref):
        dp = degp_ref[...]
        deg = dp[:PP] + dp[PP:]
        norm = lax.rsqrt(jnp.maximum(deg, 1.0))
        norm_ref[...] = norm
        h0n_ref[...] = x0_ref[...] * norm

    h0n_pack, norm_pack = pl.pallas_call(
        tc1,
        out_shape=[jax.ShapeDtypeStruct((PP, 8 * F), f32),
                   jax.ShapeDtypeStruct((PP, 8 * F), f32)],
    )(x0_pack, degp.reshape(2 * PP, 8 * F))

    agg0p = agg_k(h0n_pack.reshape(8 * PP, F), srcs, dsts, zeros_pad)

    w1p = jnp.kron(jnp.eye(8, dtype=f32), W1)
    b0p = jnp.tile(b0, 8).reshape(1, 8 * F)

    def tc2(aggp_ref, norm_ref, w1p_ref, b0p_ref, h1n_ref):
        ap = aggp_ref[...]
        nrm = norm_ref[...]
        agg0 = (ap[:PP] + ap[PP:]) * nrm
        z0 = jnp.maximum(agg0 + b0p_ref[...], 0.0)
        h1n_ref[...] = jnp.dot(z0, w1p_ref[...],
                               preferred_element_type=f32) * nrm

    h1n_pack = pl.pallas_call(
        tc2,
        out_shape=jax.ShapeDtypeStruct((PP, 8 * F), f32),
    )(agg0p.reshape(2 * PP, 8 * F), norm_pack, w1p, b0p)

    agg1p = agg_k(h1n_pack.reshape(8 * PP, F), srcs, dsts, zeros_pad)

    def tc3(aggp_ref, norm_ref, b1_ref, out_ref):
        ap = aggp_ref[...]
        agg1 = (ap[:PP] + ap[PP:]) * norm_ref[...]
        for u in range(8):
            out_ref[:, u, :] = agg1[:P, u * F:(u + 1) * F] + b1_ref[...]

    out3 = pl.pallas_call(
        tc3,
        out_shape=jax.ShapeDtypeStruct((P, 8, F), f32),
    )(agg1p.reshape(2 * PP, 8 * F), norm_pack, b1.reshape(1, F))

    return out3.reshape(N, F)


# speedup vs baseline: 31.5648x; 1.0758x over previous
import functools

import jax
import jax.numpy as jnp
from jax import lax
from jax.experimental import pallas as pl
from jax.experimental.pallas import tpu as pltpu
from jax.experimental.pallas import tpu_sc as plsc

NC = 2
NS = 16
NW = NC * NS
CH = 128
F = 16
NB = 8
LA = 4


def _sc_mesh():
    return plsc.VectorSubcoreMesh(
        core_axis_name="c", subcore_axis_name="s",
        num_cores=NC, num_subcores=NS)


def _make_deg_kernel(NPAD, CPW):
    stripe = NPAD // NS
    n_groups = CPW // NB

    def body(dsts_hbm, zeros_hbm, ones_hbm, out_hbm, dst_v, rows_v, agg_sh, ssem):
        c = lax.axis_index("c")
        s = lax.axis_index("s")
        wid = s * NC + c
        pltpu.sync_copy(zeros_hbm.at[pl.ds(s * stripe, stripe)],
                        agg_sh.at[pl.ds(s * stripe, stripe)])
        pltpu.sync_copy(ones_hbm, rows_v)
        pltpu.sync_copy(dsts_hbm.at[:, wid], dst_v)
        plsc.subcore_barrier()

        def fire(j, t):
            pltpu.async_copy(rows_v, agg_sh.at[dst_v.at[j]], ssem.at[t],
                             add=True)

        def drain(t):
            pltpu.make_async_copy(rows_v, agg_sh.at[dst_v.at[0]],
                                  ssem.at[t]).wait()

        def group(g, carry):
            for t in range(NB):
                j = g * NB + t
                @pl.when(j >= NB)
                def _():
                    drain(t)
                fire(j, t)
            return carry

        lax.fori_loop(0, n_groups, group, 0)
        for t in range(NB):
            drain(t)
        plsc.subcore_barrier()
        pltpu.sync_copy(agg_sh.at[pl.ds(s * stripe, stripe)],
                        out_hbm.at[pl.ds(c * NPAD + s * stripe, stripe)])

    return pl.kernel(
        body,
        out_type=jax.ShapeDtypeStruct((2 * NPAD, F), jnp.float32),
        mesh=_sc_mesh(),
        compiler_params=pltpu.CompilerParams(use_tc_tiling_on_sc=False),
        scratch_types=[
            pltpu.VMEM((CPW, CH), jnp.int32),
            pltpu.VMEM((CH, F), jnp.float32),
            pltpu.VMEM_SHARED((NPAD, F), jnp.float32),
            pltpu.SemaphoreType.DMA((NB,)),
        ],
    )


def _make_agg_kernel(NPAD, CPW):
    stripe = NPAD // NS
    n_groups = CPW // NB

    def body(h_hbm, srcs_hbm, dsts_hbm, zeros_hbm, out_hbm,
             src_v, dst_v, rows_v, agg_sh, gsem, ssem):
        c = lax.axis_index("c")
        s = lax.axis_index("s")
        wid = s * NC + c
        pltpu.sync_copy(zeros_hbm.at[pl.ds(s * stripe, stripe)],
                        agg_sh.at[pl.ds(s * stripe, stripe)])
        pltpu.sync_copy(srcs_hbm.at[:, wid], src_v)
        pltpu.sync_copy(dsts_hbm.at[:, wid], dst_v)
        plsc.subcore_barrier()

        def fire_gather(j, t):
            pltpu.async_copy(h_hbm.at[src_v.at[j]], rows_v.at[t], gsem.at[t])

        def wait_gather(t):
            pltpu.make_async_copy(h_hbm.at[src_v.at[0]], rows_v.at[t],
                                  gsem.at[t]).wait()

        def fire_scatter(j, t):
            pltpu.async_copy(rows_v.at[t], agg_sh.at[dst_v.at[j]], ssem.at[t],
                             add=True)

        def wait_scatter(t):
            pltpu.make_async_copy(rows_v.at[t], agg_sh.at[dst_v.at[0]],
                                  ssem.at[t]).wait()

        for t in range(LA):
            fire_gather(t, t)

        def group(g, carry):
            for t in range(NB):
                j = g * NB + t
                wait_gather(t)
                fire_scatter(j, t)
                s2 = (t + LA) % NB
                jg = j + LA

                @pl.when(jnp.logical_and(jg >= NB, jg < CPW))
                def _():
                    wait_scatter(s2)

                @pl.when(jg < CPW)
                def _():
                    fire_gather(jg, s2)
            return carry

        lax.fori_loop(0, n_groups, group, 0)
        for t in range(NB):
            wait_scatter(t)
        plsc.subcore_barrier()
        pltpu.sync_copy(agg_sh.at[pl.ds(s * stripe, stripe)],
                        out_hbm.at[pl.ds(c * NPAD + s * stripe, stripe)])

    return pl.kernel(
        body,
        out_type=jax.ShapeDtypeStruct((2 * NPAD, F), jnp.float32),
        mesh=_sc_mesh(),
        compiler_params=pltpu.CompilerParams(use_tc_tiling_on_sc=False),
        scratch_types=[
            pltpu.VMEM((CPW, CH), jnp.int32),
            pltpu.VMEM((CPW, CH), jnp.int32),
            pltpu.VMEM((NB, CH, F), jnp.float32),
            pltpu.VMEM_SHARED((NPAD, F), jnp.float32),
            pltpu.SemaphoreType.DMA((NB,)),
            pltpu.SemaphoreType.DMA((NB,)),
        ],
    )


def kernel(features, edge_index, W0, b0, W1, b1):
    N, IN_FEATS = features.shape
    E = edge_index.shape[1]
    f32 = jnp.float32

    NPAD = ((N // 256) + 1) * 256
    CPW = -(-E // (NW * CH))
    CPW = -(-CPW // NB) * NB
    EP = NW * CH * CPW
    P = N // 8
    PP = NPAD // 8

    src = edge_index[0]
    dst = edge_index[1]
    pad_i = jnp.arange(EP - E, dtype=jnp.int32)
    srcs = jnp.concatenate([src, pad_i % 128]).reshape(CPW, NW, CH)
    dsts = jnp.concatenate([dst, N + pad_i % (NPAD - N)]).reshape(CPW, NW, CH)

    zeros_pad = jnp.zeros((NPAD, F), f32)
    ones_rows = jnp.ones((CH, F), f32)

    deg_k = _make_deg_kernel(NPAD, CPW)
    agg_k = _make_agg_kernel(NPAD, CPW)

    degp = deg_k(dsts, zeros_pad, ones_rows)

    feats3 = features.reshape(P, 8, IN_FEATS)

    def tc_x0(f3_ref, w0_ref, x0_ref):
        parts = [jnp.dot(f3_ref[:, u, :], w0_ref[...],
                         preferred_element_type=f32) for u in range(8)]
        x0 = jnp.concatenate(parts, axis=1)
        x0_ref[...] = jnp.concatenate(
            [x0, jnp.zeros((PP - P, 8 * F), f32)], axis=0)

    x0_pack = pl.pallas_call(
        tc_x0,
        out_shape=jax.ShapeDtypeStruct((PP, 8 * F), f32),
    )(feats3, W0)

    def tc1(x0_ref, degp_ref, h0n_ref, norm_ref):
        dp = degp_ref[...]
        deg = dp[:PP] + dp[PP:]
        norm = lax.rsqrt(jnp.maximum(deg, 1.0))
        norm_ref[...] = norm
        h0n_ref[...] = x0_ref[...] * norm

    h0n_pack, norm_pack = pl.pallas_call(
        tc1,
        out_shape=[jax.ShapeDtypeStruct((PP, 8 * F), f32),
                   jax.ShapeDtypeStruct((PP, 8 * F), f32)],
    )(x0_pack, degp.reshape(2 * PP, 8 * F))

    agg0p = agg_k(h0n_pack.reshape(8 * PP, F), srcs, dsts, zeros_pad)

    w1p = jnp.kron(jnp.eye(8, dtype=f32), W1)
    b0p = jnp.tile(b0, 8).reshape(1, 8 * F)

    def tc2(aggp_ref, norm_ref, w1p_ref, b0p_ref, h1n_ref):
        ap = aggp_ref[...]
        nrm = norm_ref[...]
        agg0 = (ap[:PP] + ap[PP:]) * nrm
        z0 = jnp.maximum(agg0 + b0p_ref[...], 0.0)
        h1n_ref[...] = jnp.dot(z0, w1p_ref[...],
                               preferred_element_type=f32) * nrm

    h1n_pack = pl.pallas_call(
        tc2,
        out_shape=jax.ShapeDtypeStruct((PP, 8 * F), f32),
    )(agg0p.reshape(2 * PP, 8 * F), norm_pack, w1p, b0p)

    agg1p = agg_k(h1n_pack.reshape(8 * PP, F), srcs, dsts, zeros_pad)

    b1p = jnp.tile(b1, 8).reshape(1, 8 * F)

    def tc3(aggp_ref, norm_ref, b1p_ref, out_ref):
        ap = aggp_ref[...]
        agg1 = (ap[:PP] + ap[PP:]) * norm_ref[...] + b1p_ref[...]
        out_ref[...] = agg1[:P].reshape(P, 8, F)

    out3 = pl.pallas_call(
        tc3,
        out_shape=jax.ShapeDtypeStruct((P, 8, F), f32),
    )(agg1p.reshape(2 * PP, 8 * F), norm_pack, b1p)

    return out3.reshape(N, F)


# speedup vs baseline: 33.0275x; 1.0463x over previous
import functools

import jax
import jax.numpy as jnp
from jax import lax
from jax.experimental import pallas as pl
from jax.experimental.pallas import tpu as pltpu
from jax.experimental.pallas import tpu_sc as plsc

NC = 2
NS = 16
NW = NC * NS
CH = 128
F = 16
NB = 10
LA = 5


def _sc_mesh():
    return plsc.VectorSubcoreMesh(
        core_axis_name="c", subcore_axis_name="s",
        num_cores=NC, num_subcores=NS)


def _make_deg_kernel(NPAD, CPW):
    stripe = NPAD // NS
    n_groups = CPW // NB

    def body(dsts_hbm, zeros_hbm, ones_hbm, out_hbm, dst_v, rows_v, agg_sh, ssem):
        c = lax.axis_index("c")
        s = lax.axis_index("s")
        wid = s * NC + c
        pltpu.sync_copy(zeros_hbm.at[pl.ds(s * stripe, stripe)],
                        agg_sh.at[pl.ds(s * stripe, stripe)])
        pltpu.sync_copy(ones_hbm, rows_v)
        pltpu.sync_copy(dsts_hbm.at[:, wid], dst_v)
        plsc.subcore_barrier()

        def fire(j, t):
            pltpu.async_copy(rows_v, agg_sh.at[dst_v.at[j]], ssem.at[t],
                             add=True)

        def drain(t):
            pltpu.make_async_copy(rows_v, agg_sh.at[dst_v.at[0]],
                                  ssem.at[t]).wait()

        def group(g, carry):
            for t in range(NB):
                j = g * NB + t
                @pl.when(j >= NB)
                def _():
                    drain(t)
                fire(j, t)
            return carry

        lax.fori_loop(0, n_groups, group, 0)
        for t in range(NB):
            drain(t)
        plsc.subcore_barrier()
        pltpu.sync_copy(agg_sh.at[pl.ds(s * stripe, stripe)],
                        out_hbm.at[pl.ds(c * NPAD + s * stripe, stripe)])

    return pl.kernel(
        body,
        out_type=jax.ShapeDtypeStruct((2 * NPAD, F), jnp.float32),
        mesh=_sc_mesh(),
        compiler_params=pltpu.CompilerParams(use_tc_tiling_on_sc=False),
        scratch_types=[
            pltpu.VMEM((CPW, CH), jnp.int32),
            pltpu.VMEM((CH, F), jnp.float32),
            pltpu.VMEM_SHARED((NPAD, F), jnp.float32),
            pltpu.SemaphoreType.DMA((NB,)),
        ],
    )


def _make_agg_kernel(NPAD, CPW):
    stripe = NPAD // NS
    n_groups = CPW // NB

    def body(h_hbm, srcs_hbm, dsts_hbm, zeros_hbm, out_hbm,
             src_v, dst_v, rows_v, agg_sh, gsem, ssem):
        c = lax.axis_index("c")
        s = lax.axis_index("s")
        wid = s * NC + c
        pltpu.sync_copy(zeros_hbm.at[pl.ds(s * stripe, stripe)],
                        agg_sh.at[pl.ds(s * stripe, stripe)])
        pltpu.sync_copy(srcs_hbm.at[:, wid], src_v)
        pltpu.sync_copy(dsts_hbm.at[:, wid], dst_v)
        plsc.subcore_barrier()

        def fire_gather(j, t):
            pltpu.async_copy(h_hbm.at[src_v.at[j]], rows_v.at[t], gsem.at[t])

        def wait_gather(t):
            pltpu.make_async_copy(h_hbm.at[src_v.at[0]], rows_v.at[t],
                                  gsem.at[t]).wait()

        def fire_scatter(j, t):
            pltpu.async_copy(rows_v.at[t], agg_sh.at[dst_v.at[j]], ssem.at[t],
                             add=True)

        def wait_scatter(t):
            pltpu.make_async_copy(rows_v.at[t], agg_sh.at[dst_v.at[0]],
                                  ssem.at[t]).wait()

        for t in range(LA):
            fire_gather(t, t)

        def group(g, carry):
            for t in range(NB):
                j = g * NB + t
                wait_gather(t)
                fire_scatter(j, t)
                s2 = (t + LA) % NB
                jg = j + LA

                @pl.when(jnp.logical_and(jg >= NB, jg < CPW))
                def _():
                    wait_scatter(s2)

                @pl.when(jg < CPW)
                def _():
                    fire_gather(jg, s2)
            return carry

        lax.fori_loop(0, n_groups, group, 0)
        for t in range(NB):
            wait_scatter(t)
        plsc.subcore_barrier()
        pltpu.sync_copy(agg_sh.at[pl.ds(s * stripe, stripe)],
                        out_hbm.at[pl.ds(c * NPAD + s * stripe, stripe)])

    return pl.kernel(
        body,
        out_type=jax.ShapeDtypeStruct((2 * NPAD, F), jnp.float32),
        mesh=_sc_mesh(),
        compiler_params=pltpu.CompilerParams(use_tc_tiling_on_sc=False),
        scratch_types=[
            pltpu.VMEM((CPW, CH), jnp.int32),
            pltpu.VMEM((CPW, CH), jnp.int32),
            pltpu.VMEM((NB, CH, F), jnp.float32),
            pltpu.VMEM_SHARED((NPAD, F), jnp.float32),
            pltpu.SemaphoreType.DMA((NB,)),
            pltpu.SemaphoreType.DMA((NB,)),
        ],
    )


def kernel(features, edge_index, W0, b0, W1, b1):
    N, IN_FEATS = features.shape
    E = edge_index.shape[1]
    f32 = jnp.float32

    NPAD = ((N // 256) + 1) * 256
    CPW = -(-E // (NW * CH))
    CPW = -(-CPW // NB) * NB
    EP = NW * CH * CPW
    P = N // 8
    PP = NPAD // 8

    src = edge_index[0]
    dst = edge_index[1]
    pad_i = jnp.arange(EP - E, dtype=jnp.int32)
    srcs = jnp.concatenate([src, pad_i % 128]).reshape(CPW, NW, CH)
    dsts = jnp.concatenate([dst, N + pad_i % (NPAD - N)]).reshape(CPW, NW, CH)

    zeros_pad = jnp.zeros((NPAD, F), f32)
    ones_rows = jnp.ones((CH, F), f32)

    deg_k = _make_deg_kernel(NPAD, CPW)
    agg_k = _make_agg_kernel(NPAD, CPW)

    degp = deg_k(dsts, zeros_pad, ones_rows)

    feats3 = features.reshape(P, 8, IN_FEATS)

    def tc_x0(f3_ref, w0_ref, x0_ref):
        parts = [jnp.dot(f3_ref[:, u, :], w0_ref[...],
                         preferred_element_type=f32) for u in range(8)]
        x0 = jnp.concatenate(parts, axis=1)
        x0_ref[...] = jnp.concatenate(
            [x0, jnp.zeros((PP - P, 8 * F), f32)], axis=0)

    x0_pack = pl.pallas_call(
        tc_x0,
        out_shape=jax.ShapeDtypeStruct((PP, 8 * F), f32),
    )(feats3, W0)

    def tc1(x0_ref, degp_ref, h0n_ref, norm_ref):
        dp = degp_ref[...]
        deg = dp[:PP] + dp[PP:]
        norm = lax.rsqrt(jnp.maximum(deg, 1.0))
        norm_ref[...] = norm
        h0n_ref[...] = x0_ref[...] * norm

    h0n_pack, norm_pack = pl.pallas_call(
        tc1,
        out_shape=[jax.ShapeDtypeStruct((PP, 8 * F), f32),
                   jax.ShapeDtypeStruct((PP, 8 * F), f32)],
    )(x0_pack, degp.reshape(2 * PP, 8 * F))

    agg0p = agg_k(h0n_pack.reshape(8 * PP, F), srcs, dsts, zeros_pad)

    w1p = jnp.kron(jnp.eye(8, dtype=f32), W1)
    b0p = jnp.tile(b0, 8).reshape(1, 8 * F)

    def tc2(aggp_ref, norm_ref, w1p_ref, b0p_ref, h1n_ref):
        ap = aggp_ref[...]
        nrm = norm_ref[...]
        agg0 = (ap[:PP] + ap[PP:]) * nrm
        z0 = jnp.maximum(agg0 + b0p_ref[...], 0.0)
        h1n_ref[...] = jnp.dot(z0, w1p_ref[...],
                               preferred_element_type=f32) * nrm

    h1n_pack = pl.pallas_call(
        tc2,
        out_shape=jax.ShapeDtypeStruct((PP, 8 * F), f32),
    )(agg0p.reshape(2 * PP, 8 * F), norm_pack, w1p, b0p)

    agg1p = agg_k(h1n_pack.reshape(8 * PP, F), srcs, dsts, zeros_pad)

    b1p = jnp.tile(b1, 8).reshape(1, 8 * F)

    def tc3(aggp_ref, norm_ref, b1p_ref, out_ref):
        ap = aggp_ref[...]
        agg1 = (ap[:PP] + ap[PP:]) * norm_ref[...] + b1p_ref[...]
        out_ref[...] = agg1[:P].reshape(P, 8, F)

    out3 = pl.pallas_call(
        tc3,
        out_shape=jax.ShapeDtypeStruct((P, 8, F), f32),
    )(agg1p.reshape(2 * PP, 8 * F), norm_pack, b1p)

    return out3.reshape(N, F)


# speedup vs baseline: 34.2763x; 1.0378x over previous
import functools

import jax
import jax.numpy as jnp
from jax import lax
from jax.experimental import pallas as pl
from jax.experimental.pallas import tpu as pltpu
from jax.experimental.pallas import tpu_sc as plsc

NC = 2
NS = 16
NW = NC * NS
CH = 128
F = 16
NB = 10
LA = 5


def _sc_mesh():
    return plsc.VectorSubcoreMesh(
        core_axis_name="c", subcore_axis_name="s",
        num_cores=NC, num_subcores=NS)


def _make_deg_kernel(NPAD, CPW):
    stripe = NPAD // NS
    n_groups = CPW // NB

    def body(dsts_hbm, zeros_hbm, ones_hbm, out_hbm, dst_v, rows_v, agg_sh, ssem):
        c = lax.axis_index("c")
        s = lax.axis_index("s")
        wid = s * NC + c
        z = pltpu.async_copy(zeros_hbm.at[pl.ds(s * stripe, stripe)],
                             agg_sh.at[pl.ds(s * stripe, stripe)], ssem.at[NB - 1])
        a = pltpu.async_copy(ones_hbm, rows_v, ssem.at[NB - 2])
        b = pltpu.async_copy(dsts_hbm.at[:, wid], dst_v, ssem.at[NB - 3])
        z.wait()
        a.wait()
        b.wait()
        plsc.subcore_barrier()

        def fire(j, t):
            pltpu.async_copy(rows_v, agg_sh.at[dst_v.at[j]], ssem.at[t],
                             add=True)

        def drain(t):
            pltpu.make_async_copy(rows_v, agg_sh.at[dst_v.at[0]],
                                  ssem.at[t]).wait()

        def group(g, carry):
            for t in range(NB):
                j = g * NB + t
                @pl.when(j >= NB)
                def _():
                    drain(t)
                fire(j, t)
            return carry

        lax.fori_loop(0, n_groups, group, 0)
        for t in range(NB):
            drain(t)
        plsc.subcore_barrier()
        pltpu.sync_copy(agg_sh.at[pl.ds(s * stripe, stripe)],
                        out_hbm.at[pl.ds(c * NPAD + s * stripe, stripe)])

    return pl.kernel(
        body,
        out_type=jax.ShapeDtypeStruct((2 * NPAD, F), jnp.float32),
        mesh=_sc_mesh(),
        compiler_params=pltpu.CompilerParams(use_tc_tiling_on_sc=False),
        scratch_types=[
            pltpu.VMEM((CPW, CH), jnp.int32),
            pltpu.VMEM((CH, F), jnp.float32),
            pltpu.VMEM_SHARED((NPAD, F), jnp.float32),
            pltpu.SemaphoreType.DMA((NB,)),
        ],
    )


def _make_agg_kernel(NPAD, CPW):
    stripe = NPAD // NS
    n_groups = CPW // NB

    def body(h_hbm, srcs_hbm, dsts_hbm, zeros_hbm, out_hbm,
             src_v, dst_v, rows_v, agg_sh, gsem, ssem):
        c = lax.axis_index("c")
        s = lax.axis_index("s")
        wid = s * NC + c
        z = pltpu.async_copy(zeros_hbm.at[pl.ds(s * stripe, stripe)],
                             agg_sh.at[pl.ds(s * stripe, stripe)], gsem.at[NB - 1])
        a = pltpu.async_copy(srcs_hbm.at[:, wid], src_v, ssem.at[NB - 1])
        b = pltpu.async_copy(dsts_hbm.at[:, wid], dst_v, ssem.at[NB - 2])
        z.wait()
        a.wait()
        b.wait()
        plsc.subcore_barrier()

        def fire_gather(j, t):
            pltpu.async_copy(h_hbm.at[src_v.at[j]], rows_v.at[t], gsem.at[t])

        def wait_gather(t):
            pltpu.make_async_copy(h_hbm.at[src_v.at[0]], rows_v.at[t],
                                  gsem.at[t]).wait()

        def fire_scatter(j, t):
            pltpu.async_copy(rows_v.at[t], agg_sh.at[dst_v.at[j]], ssem.at[t],
                             add=True)

        def wait_scatter(t):
            pltpu.make_async_copy(rows_v.at[t], agg_sh.at[dst_v.at[0]],
                                  ssem.at[t]).wait()

        for t in range(LA):
            fire_gather(t, t)

        def group(g, carry):
            for t in range(NB):
                j = g * NB + t
                wait_gather(t)
                fire_scatter(j, t)
                s2 = (t + LA) % NB
                jg = j + LA

                @pl.when(jnp.logical_and(jg >= NB, jg < CPW))
                def _():
                    wait_scatter(s2)

                @pl.when(jg < CPW)
                def _():
                    fire_gather(jg, s2)
            return carry

        lax.fori_loop(0, n_groups, group, 0)
        for t in range(NB):
            wait_scatter(t)
        plsc.subcore_barrier()
        pltpu.sync_copy(agg_sh.at[pl.ds(s * stripe, stripe)],
                        out_hbm.at[pl.ds(c * NPAD + s * stripe, stripe)])

    return pl.kernel(
        body,
        out_type=jax.ShapeDtypeStruct((2 * NPAD, F), jnp.float32),
        mesh=_sc_mesh(),
        compiler_params=pltpu.CompilerParams(use_tc_tiling_on_sc=False),
        scratch_types=[
            pltpu.VMEM((CPW, CH), jnp.int32),
            pltpu.VMEM((CPW, CH), jnp.int32),
            pltpu.VMEM((NB, CH, F), jnp.float32),
            pltpu.VMEM_SHARED((NPAD, F), jnp.float32),
            pltpu.SemaphoreType.DMA((NB,)),
            pltpu.SemaphoreType.DMA((NB,)),
        ],
    )


def kernel(features, edge_index, W0, b0, W1, b1):
    N, IN_FEATS = features.shape
    E = edge_index.shape[1]
    f32 = jnp.float32

    NPAD = ((N // 256) + 1) * 256
    CPW = -(-E // (NW * CH))
    CPW = -(-CPW // NB) * NB
    EP = NW * CH * CPW
    P = N // 8
    PP = NPAD // 8

    src = edge_index[0]
    dst = edge_index[1]
    pad_i = jnp.arange(EP - E, dtype=jnp.int32)
    srcs = jnp.concatenate([src, pad_i % 128]).reshape(CPW, NW, CH)
    dsts = jnp.concatenate([dst, N + pad_i % (NPAD - N)]).reshape(CPW, NW, CH)

    zeros_pad = jnp.zeros((NPAD, F), f32)
    ones_rows = jnp.ones((CH, F), f32)

    deg_k = _make_deg_kernel(NPAD, CPW)
    agg_k = _make_agg_kernel(NPAD, CPW)

    degp = deg_k(dsts, zeros_pad, ones_rows)

    feats3 = features.reshape(P, 8, IN_FEATS)

    def tc_x0(f3_ref, w0_ref, x0_ref):
        parts = [jnp.dot(f3_ref[:, u, :], w0_ref[...],
                         preferred_element_type=f32) for u in range(8)]
        x0 = jnp.concatenate(parts, axis=1)
        x0_ref[...] = jnp.concatenate(
            [x0, jnp.zeros((PP - P, 8 * F), f32)], axis=0)

    x0_pack = pl.pallas_call(
        tc_x0,
        out_shape=jax.ShapeDtypeStruct((PP, 8 * F), f32),
    )(feats3, W0)

    def tc1(x0_ref, degp_ref, h0n_ref, norm_ref):
        dp = degp_ref[...]
        deg = dp[:PP] + dp[PP:]
        norm = lax.rsqrt(jnp.maximum(deg, 1.0))
        norm_ref[...] = norm
        h0n_ref[...] = x0_ref[...] * norm

    h0n_pack, norm_pack = pl.pallas_call(
        tc1,
        out_shape=[jax.ShapeDtypeStruct((PP, 8 * F), f32),
                   jax.ShapeDtypeStruct((PP, 8 * F), f32)],
    )(x0_pack, degp.reshape(2 * PP, 8 * F))

    agg0p = agg_k(h0n_pack.reshape(8 * PP, F), srcs, dsts, zeros_pad)

    w1p = jnp.kron(jnp.eye(8, dtype=f32), W1)
    b0p = jnp.tile(b0, 8).reshape(1, 8 * F)

    def tc2(aggp_ref, norm_ref, w1p_ref, b0p_ref, h1n_ref):
        ap = aggp_ref[...]
        nrm = norm_ref[...]
        agg0 = (ap[:PP] + ap[PP:]) * nrm
        z0 = jnp.maximum(agg0 + b0p_ref[...], 0.0)
        h1n_ref[...] = jnp.dot(z0, w1p_ref[...],
                               preferred_element_type=f32) * nrm

    h1n_pack = pl.pallas_call(
        tc2,
        out_shape=jax.ShapeDtypeStruct((PP, 8 * F), f32),
    )(agg0p.reshape(2 * PP, 8 * F), norm_pack, w1p, b0p)

    agg1p = agg_k(h1n_pack.reshape(8 * PP, F), srcs, dsts, zeros_pad)

    b1p = jnp.tile(b1, 8).reshape(1, 8 * F)

    def tc3(aggp_ref, norm_ref, b1p_ref, out_ref):
        ap = aggp_ref[...]
        agg1 = (ap[:PP] + ap[PP:]) * norm_ref[...] + b1p_ref[...]
        out_ref[...] = agg1[:P].reshape(P, 8, F)

    out3 = pl.pallas_call(
        tc3,
        out_shape=jax.ShapeDtypeStruct((P, 8, F), f32),
    )(agg1p.reshape(2 * PP, 8 * F), norm_pack, b1p)

    return out3.reshape(N, F)


# speedup vs baseline: 34.3343x; 1.0017x over previous
import functools

import jax
import jax.numpy as jnp
from jax import lax
from jax.experimental import pallas as pl
from jax.experimental.pallas import tpu as pltpu
from jax.experimental.pallas import tpu_sc as plsc

NC = 2
NS = 16
NW = NC * NS
CH = 128
F = 16
NB = 10
LA = 5


def _sc_mesh():
    return plsc.VectorSubcoreMesh(
        core_axis_name="c", subcore_axis_name="s",
        num_cores=NC, num_subcores=NS)


def _make_deg_kernel(NPAD, CPW):
    stripe = NPAD // NS
    n_groups = CPW // NB

    def body(dsts_hbm, zeros_hbm, ones_hbm, out_hbm, dst_v, rows_v, agg_sh, ssem):
        c = lax.axis_index("c")
        s = lax.axis_index("s")
        wid = s * NC + c
        z = pltpu.async_copy(zeros_hbm.at[pl.ds(s * stripe, stripe)],
                             agg_sh.at[pl.ds(s * stripe, stripe)], ssem.at[NB - 1])
        a = pltpu.async_copy(ones_hbm, rows_v, ssem.at[NB - 2])
        b = pltpu.async_copy(dsts_hbm.at[:, wid], dst_v, ssem.at[NB - 3])
        z.wait()
        a.wait()
        b.wait()
        plsc.subcore_barrier()

        def fire(j, t):
            pltpu.async_copy(rows_v, agg_sh.at[dst_v.at[j]], ssem.at[t],
                             add=True)

        def drain(t):
            pltpu.make_async_copy(rows_v, agg_sh.at[dst_v.at[0]],
                                  ssem.at[t]).wait()

        def group(g, carry):
            for t in range(NB):
                j = g * NB + t
                @pl.when(j >= NB)
                def _():
                    drain(t)
                fire(j, t)
            return carry

        lax.fori_loop(0, n_groups, group, 0)
        for t in range(NB):
            drain(t)
        plsc.subcore_barrier()
        pltpu.sync_copy(agg_sh.at[pl.ds(s * stripe, stripe)],
                        out_hbm.at[pl.ds(c * NPAD + s * stripe, stripe)])

    return pl.kernel(
        body,
        out_type=jax.ShapeDtypeStruct((2 * NPAD, F), jnp.float32),
        mesh=_sc_mesh(),
        compiler_params=pltpu.CompilerParams(use_tc_tiling_on_sc=False),
        scratch_types=[
            pltpu.VMEM((CPW, CH), jnp.int32),
            pltpu.VMEM((CH, F), jnp.float32),
            pltpu.VMEM_SHARED((NPAD, F), jnp.float32),
            pltpu.SemaphoreType.DMA((NB,)),
        ],
    )


def _make_agg_kernel(NPAD, CPW):
    stripe = NPAD // NS
    n_groups = CPW // NB

    def body(h_hbm, srcs_hbm, dsts_hbm, zeros_hbm, out_hbm,
             src_v, dst_v, rows_v, agg_sh, gsem, ssem):
        c = lax.axis_index("c")
        s = lax.axis_index("s")
        wid = s * NC + c
        z = pltpu.async_copy(zeros_hbm.at[pl.ds(s * stripe, stripe)],
                             agg_sh.at[pl.ds(s * stripe, stripe)], gsem.at[NB - 1])
        a = pltpu.async_copy(srcs_hbm.at[:, wid], src_v, ssem.at[NB - 1])
        b = pltpu.async_copy(dsts_hbm.at[:, wid], dst_v, ssem.at[NB - 2])
        z.wait()
        a.wait()
        b.wait()
        plsc.subcore_barrier()

        def fire_gather(j, t):
            pltpu.async_copy(h_hbm.at[src_v.at[j]], rows_v.at[t], gsem.at[t])

        def wait_gather(t):
            pltpu.make_async_copy(h_hbm.at[src_v.at[0]], rows_v.at[t],
                                  gsem.at[t]).wait()

        def fire_scatter(j, t):
            pltpu.async_copy(rows_v.at[t], agg_sh.at[dst_v.at[j]], ssem.at[t],
                             add=True)

        def wait_scatter(t):
            pltpu.make_async_copy(rows_v.at[t], agg_sh.at[dst_v.at[0]],
                                  ssem.at[t]).wait()

        for t in range(LA):
            fire_gather(t, t)

        def group(g, carry):
            for t in range(NB):
                j = g * NB + t
                wait_gather(t)
                fire_scatter(j, t)
                s2 = (t + LA) % NB
                jg = j + LA

                @pl.when(jnp.logical_and(jg >= NB, jg < CPW))
                def _():
                    wait_scatter(s2)

                @pl.when(jg < CPW)
                def _():
                    fire_gather(jg, s2)
            return carry

        lax.fori_loop(0, n_groups, group, 0)
        for t in range(NB):
            wait_scatter(t)
        plsc.subcore_barrier()
        pltpu.sync_copy(agg_sh.at[pl.ds(s * stripe, stripe)],
                        out_hbm.at[pl.ds(c * NPAD + s * stripe, stripe)])

    return pl.kernel(
        body,
        out_type=jax.ShapeDtypeStruct((2 * NPAD, F), jnp.float32),
        mesh=_sc_mesh(),
        compiler_params=pltpu.CompilerParams(use_tc_tiling_on_sc=False),
        scratch_types=[
            pltpu.VMEM((CPW, CH), jnp.int32),
            pltpu.VMEM((CPW, CH), jnp.int32),
            pltpu.VMEM((NB, CH, F), jnp.float32),
            pltpu.VMEM_SHARED((NPAD, F), jnp.float32),
            pltpu.SemaphoreType.DMA((NB,)),
            pltpu.SemaphoreType.DMA((NB,)),
        ],
    )


def kernel(features, edge_index, W0, b0, W1, b1):
    N, IN_FEATS = features.shape
    E = edge_index.shape[1]
    f32 = jnp.float32

    NPAD = ((N // 256) + 1) * 256
    CPW = -(-E // (NW * CH))
    CPW = -(-CPW // NB) * NB
    EP = NW * CH * CPW
    P = N // 8
    PP = NPAD // 8

    src = edge_index[0]
    dst = edge_index[1]
    pad_i = jnp.arange(EP - E, dtype=jnp.int32)
    srcs = lax.optimization_barrier(
        jnp.concatenate([src, pad_i % 128])).reshape(CPW, NW, CH)
    dsts = jnp.concatenate([dst, N + pad_i % (NPAD - N)]).reshape(CPW, NW, CH)

    zeros_pad = jnp.zeros((NPAD, F), f32)
    ones_rows = jnp.ones((CH, F), f32)

    deg_k = _make_deg_kernel(NPAD, CPW)
    agg_k = _make_agg_kernel(NPAD, CPW)

    degp = deg_k(dsts, zeros_pad, ones_rows)

    feats3 = features.reshape(P, 8, IN_FEATS)

    def tc_x0(f3_ref, w0_ref, x0_ref):
        parts = [jnp.dot(f3_ref[:, u, :], w0_ref[...],
                         preferred_element_type=f32) for u in range(8)]
        x0 = jnp.concatenate(parts, axis=1)
        x0_ref[...] = jnp.concatenate(
            [x0, jnp.zeros((PP - P, 8 * F), f32)], axis=0)

    x0_pack = pl.pallas_call(
        tc_x0,
        out_shape=jax.ShapeDtypeStruct((PP, 8 * F), f32),
    )(feats3, W0)

    def tc1(x0_ref, degp_ref, h0n_ref, norm_ref):
        dp = degp_ref[...]
        deg = dp[:PP] + dp[PP:]
        norm = lax.rsqrt(jnp.maximum(deg, 1.0))
        norm_ref[...] = norm
        h0n_ref[...] = x0_ref[...] * norm

    h0n_pack, norm_pack = pl.pallas_call(
        tc1,
        out_shape=[jax.ShapeDtypeStruct((PP, 8 * F), f32),
                   jax.ShapeDtypeStruct((PP, 8 * F), f32)],
    )(x0_pack, degp.reshape(2 * PP, 8 * F))

    agg0p = agg_k(h0n_pack.reshape(8 * PP, F), srcs, dsts, zeros_pad)

    w1p = jnp.kron(jnp.eye(8, dtype=f32), W1)
    b0p = jnp.tile(b0, 8).reshape(1, 8 * F)

    def tc2(aggp_ref, norm_ref, w1p_ref, b0p_ref, h1n_ref):
        ap = aggp_ref[...]
        nrm = norm_ref[...]
        agg0 = (ap[:PP] + ap[PP:]) * nrm
        z0 = jnp.maximum(agg0 + b0p_ref[...], 0.0)
        h1n_ref[...] = jnp.dot(z0, w1p_ref[...],
                               preferred_element_type=f32) * nrm

    h1n_pack = pl.pallas_call(
        tc2,
        out_shape=jax.ShapeDtypeStruct((PP, 8 * F), f32),
    )(agg0p.reshape(2 * PP, 8 * F), norm_pack, w1p, b0p)

    agg1p = agg_k(h1n_pack.reshape(8 * PP, F), srcs, dsts, zeros_pad)

    b1p = jnp.tile(b1, 8).reshape(1, 8 * F)

    def tc3(aggp_ref, norm_ref, b1p_ref, out_ref):
        ap = aggp_ref[...]
        agg1 = (ap[:PP] + ap[PP:]) * norm_ref[...] + b1p_ref[...]
        out_ref[...] = agg1[:P].reshape(P, 8, F)

    out3 = pl.pallas_call(
        tc3,
        out_shape=jax.ShapeDtypeStruct((P, 8, F), f32),
    )(agg1p.reshape(2 * PP, 8 * F), norm_pack, b1p)

    return out3.reshape(N, F)
